# baseline (speedup 1.0000x reference)
import functools

import jax
import jax.numpy as jnp
from jax import lax
from jax.experimental import pallas as pl
from jax.experimental.pallas import tpu as pltpu
from jax.experimental.pallas import tpu_sc as plsc

T, B, D, L = 2048, 4, 1024, 4
TB = 128

_SC_INFO = plsc.get_sparse_core_info()
NC, NS = _SC_INFO.num_cores, _SC_INFO.num_subcores
NW = NC * NS
TGROUPS = NW // B
PER_W = T // TGROUPS
NR = 16
CHUNKS = PER_W // NR


T_TC = 1280
T_SC = T - T_TC
NR2 = 8
ROWS_W = T_SC // NW
GCHUNKS = ROWS_W // NR2


def _gate_kernel(h0_ref, h1_ref, h2_ref, h3_ref, wg_ref, gp_ref, acc_ref):
    i = pl.program_id(0)
    nsteps = pl.num_programs(0)

    @pl.when(i == 0)
    def _init():
        acc_ref[...] = jnp.zeros_like(acc_ref)

    for l, h_ref in enumerate((h0_ref, h1_ref, h2_ref, h3_ref)):
        acc_ref[l] += jnp.sum(h_ref[...], axis=0)

    @pl.when(i == nsteps - 1)
    def _finish():
        wgv = wg_ref[...]
        colsum = acc_ref[...]
        gate_lb = jnp.sum(colsum * wgv[None], axis=2)
        gp_ref[...] = lax.pad(gate_lb, 0.0, ((0, 8 - L, 0), (0, 128 - B, 0)))


def _finalize_kernel(gp_ref, psc_ref, mi_ref, mw_ref):
    tot = None
    for w in range(NW):
        blk = psc_ref[w * 16:(w + 1) * 16, :]
        tot = blk if tot is None else tot + blk
    lane_sum = jnp.sum(tot, axis=1, keepdims=True)
    gsc = lane_sum.reshape(L, B)
    g4 = (gp_ref[0:L, 0:B] + gsc) * (1.0 / T)
    neg = jnp.float32(-jnp.inf)
    g = lax.pad(g4, neg, ((0, 8 - L, 0), (0, 128 - B, 0)))
    rows = lax.broadcasted_iota(jnp.int32, g.shape, 0)
    m1 = jnp.max(g, axis=0, keepdims=True)
    i1 = jnp.min(jnp.where(g == m1, rows, L + 4), axis=0, keepdims=True)
    g2 = jnp.where(rows == i1, neg, g)
    m2 = jnp.max(g2, axis=0, keepdims=True)
    i2 = jnp.min(jnp.where(g2 == m2, rows, L + 4), axis=0, keepdims=True)
    e2 = jnp.exp(m2 - m1)
    w1 = 0.5 / (1.0 + e2)
    w2 = (0.5 * e2) / (1.0 + e2)
    mi_ref[...] = jnp.where(rows == 0, i1, jnp.where(rows == 1, i2, 0))
    mw_ref[...] = jnp.where(rows == 0, w1, jnp.where(rows == 1, w2, 0.0))


_sc_mesh = plsc.VectorSubcoreMesh(core_axis_name="c", subcore_axis_name="s")


@functools.partial(
    pl.kernel,
    out_type=jax.ShapeDtypeStruct((NW * 16, 16), jnp.float32),
    mesh=_sc_mesh,
    scratch_types=[
        pltpu.VMEM((D,), jnp.float32),
        pltpu.VMEM((16, 16), jnp.float32),
        pltpu.VMEM((NR2, B, D), jnp.float32),
        pltpu.VMEM((NR2, B, D), jnp.float32),
        pltpu.SemaphoreType.DMA,
        pltpu.SemaphoreType.DMA,
    ],
)
def _sc_gate(wg_hbm, h0_hbm, h1_hbm, h2_hbm, h3_hbm, psc_hbm,
             wg_v, acc_v, buf0, buf1, sem0, sem1):
    wid = lax.axis_index("s") * NC + lax.axis_index("c")
    base = T_TC + wid * ROWS_W
    pltpu.sync_copy(wg_hbm, wg_v)
    hs = (h0_hbm, h1_hbm, h2_hbm, h3_hbm)
    bufs = (buf0, buf1)
    sems = (sem0, sem1)
    NK = GCHUNKS * L

    for row in range(L * B):
        acc_v[row, pl.ds(0, 16)] = jnp.zeros((16,), jnp.float32)

    def issue(l, c):
        pltpu.async_copy(hs[l].at[pl.ds(base + c * NR2, NR2)],
                         bufs[l % 2], sems[l % 2])

    issue(0, 0)
    issue(1, 0)

    @pl.loop(0, GCHUNKS)
    def _(c):
        for l in range(L):
            k = c * L + l
            buf, sem = bufs[l % 2], sems[l % 2]
            pltpu.make_async_copy(hs[0].at[pl.ds(base, NR2)], buf, sem).wait()
            for b_ in range(B):
                slab_acc = jnp.zeros((16,), jnp.float32)
                for r in range(NR2):
                    @plsc.parallel_loop(0, D // 16, unroll=8, carry=slab_acc)
                    def slab_acc(j, acc, buf=buf, b_=b_, r=r):
                        return acc + (buf[r, b_, pl.ds(j * 16, 16)]
                                      * wg_v[pl.ds(j * 16, 16)])
                row = l * B + b_
                acc_v[row, pl.ds(0, 16)] = acc_v[row, pl.ds(0, 16)] + slab_acc

            @pl.when(k + 2 < NK)
            def _(l=l, c=c):
                l2 = (l + 2) % L
                c2 = c + (1 if l >= L - 2 else 0)
                pltpu.async_copy(hs[l2].at[pl.ds(base + c2 * NR2, NR2)],
                                 bufs[l2 % 2], sems[l2 % 2])

    pltpu.sync_copy(acc_v, psc_hbm.at[pl.ds(wid * 16, 16)])


@functools.partial(
    pl.kernel,
    out_type=jax.ShapeDtypeStruct((T, B, D), jnp.float32),
    mesh=_sc_mesh,
    scratch_types=[
        pltpu.VMEM((128,), jnp.int32),
        pltpu.VMEM((128,), jnp.int32),
        pltpu.VMEM((128,), jnp.float32),
        pltpu.VMEM((128,), jnp.float32),
        pltpu.VMEM((NR, D), jnp.float32),
        pltpu.VMEM((NR, D), jnp.float32),
        pltpu.VMEM((NR, D), jnp.float32),
        pltpu.VMEM((NR, D), jnp.float32),
        pltpu.VMEM((NR, D), jnp.float32),
        pltpu.VMEM((NR, D), jnp.float32),
        pltpu.SemaphoreType.DMA,
        pltpu.SemaphoreType.DMA,
        pltpu.SemaphoreType.DMA,
    ],
)
def _sc_combine(mi_hbm, mw_hbm, h0_hbm, h1_hbm, h2_hbm, h3_hbm, out_hbm,
                ia_v, ib_v, wa_v, wb_v, a0, b0, a1, b1, a2, b2,
                sem0, sem1, sem2):
    wid = lax.axis_index("s") * NC + lax.axis_index("c")
    b = wid % B
    tg = wid // B
    base = tg * PER_W
    pltpu.sync_copy(mi_hbm.at[0], ia_v)
    pltpu.sync_copy(mi_hbm.at[1], ib_v)
    pltpu.sync_copy(mw_hbm.at[0], wa_v)
    pltpu.sync_copy(mw_hbm.at[1], wb_v)
    sA = ia_v[pl.ds(b, 16)][0]
    sB = ib_v[pl.ds(b, 16)][0]
    wA = wa_v[pl.ds(b, 16)][0]
    wB = wb_v[pl.ds(b, 16)][0]
    hs = (h0_hbm, h1_hbm, h2_hbm, h3_hbm)

    def issue(t0, bufa, bufb, sem):
        for l in range(L):
            @pl.when(sA == l)
            def _(l=l):
                pltpu.async_copy(hs[l].at[pl.ds(t0, NR), b], bufa, sem)

            @pl.when(sB == l)
            def _(l=l):
                pltpu.async_copy(hs[l].at[pl.ds(t0, NR), b], bufb, sem)

    def drain(bufa, bufb, sem):
        pltpu.make_async_copy(h0_hbm.at[pl.ds(0, NR), b], bufa, sem).wait()
        pltpu.make_async_copy(h0_hbm.at[pl.ds(0, NR), b], bufb, sem).wait()

    def compute(bufa, bufb):
        for r in range(NR):
            @plsc.parallel_loop(0, D // 16, unroll=8)
            def _(j, r=r):
                a = bufa[r, pl.ds(j * 16, 16)]
                bv = bufb[r, pl.ds(j * 16, 16)]
                bufa[r, pl.ds(j * 16, 16)] = wA * a + wB * bv

    slots = ((a0, b0, sem0), (a1, b1, sem1), (a2, b2, sem2))
    issue(base, a0, b0, sem0)
    issue(base + NR, a1, b1, sem1)

    @pl.loop(0, ((CHUNKS + 2) // 3) * 3, step=3)
    def _(c0):
        for s in range(3):
            c = c0 + s
            sa, sb, sem = slots[s]
            na, nb, nsem = slots[(s + 2) % 3]

            @pl.when(c + 2 < CHUNKS)
            def _(na=na, nb=nb, nsem=nsem, c=c):
                issue(base + (c + 2) * NR, na, nb, nsem)

            @pl.when(c < CHUNKS)
            def _(sa=sa, sb=sb, sem=sem, c=c):
                drain(sa, sb, sem)
                compute(sa, sb)
                pltpu.sync_copy(sa, out_hbm.at[pl.ds(base + c * NR, NR), b])


def kernel(h0, h1, h2, h3, Wg, bg):
    del bg
    wg2 = Wg.reshape(1, D)
    h_spec = pl.BlockSpec((TB, B, D), lambda i: (i, 0, 0))
    meta_spec = pl.BlockSpec((8, 128), lambda i: (0, 0))
    psc = _sc_gate(Wg.reshape(D), h0, h1, h2, h3)
    gp = pl.pallas_call(
        _gate_kernel,
        grid=(T_TC // TB,),
        in_specs=[h_spec, h_spec, h_spec, h_spec,
                  pl.BlockSpec((1, D), lambda i: (0, 0))],
        out_specs=meta_spec,
        out_shape=jax.ShapeDtypeStruct((8, 128), jnp.float32),
        scratch_shapes=[pltpu.VMEM((L, B, D), jnp.float32)],
    )(h0, h1, h2, h3, wg2)
    mi, mw = pl.pallas_call(
        _finalize_kernel,
        grid=(1,),
        in_specs=[pl.BlockSpec((8, 128), lambda i: (0, 0)),
                  pl.BlockSpec((NW * 16, 16), lambda i: (0, 0))],
        out_specs=[meta_spec, meta_spec],
        out_shape=[jax.ShapeDtypeStruct((8, 128), jnp.int32),
                   jax.ShapeDtypeStruct((8, 128), jnp.float32)],
    )(gp, psc)
    return _sc_combine(mi, mw, h0, h1, h2, h3)


# speedup vs baseline: 1.3548x; 1.3548x over previous
import functools

import jax
import jax.numpy as jnp
from jax import lax
from jax.experimental import pallas as pl
from jax.experimental.pallas import tpu as pltpu
from jax.experimental.pallas import tpu_sc as plsc

T, B, D, L = 2048, 4, 1024, 4
TB = 128

_SC_INFO = plsc.get_sparse_core_info()
NC, NS = _SC_INFO.num_cores, _SC_INFO.num_subcores
NW = NC * NS
TGROUPS = NW // B
PER_W = T // TGROUPS
NR = 16
CHUNKS = PER_W // NR


T_TC = 1280
T_SC = T - T_TC
NR2 = 8
ROWS_W = T_SC // NW
GCHUNKS = ROWS_W // NR2


def _gate_kernel(h0_ref, h1_ref, h2_ref, h3_ref, wg_ref, gp_ref, acc_ref):
    i = pl.program_id(0)
    nsteps = pl.num_programs(0)

    @pl.when(i == 0)
    def _init():
        acc_ref[...] = jnp.zeros_like(acc_ref)

    for l, h_ref in enumerate((h0_ref, h1_ref, h2_ref, h3_ref)):
        acc_ref[l] += jnp.sum(h_ref[...], axis=0)

    @pl.when(i == nsteps - 1)
    def _finish():
        wgv = wg_ref[...]
        colsum = acc_ref[...]
        gate_lb = jnp.sum(colsum * wgv[None], axis=2)
        gp_ref[...] = lax.pad(gate_lb, 0.0, ((0, 8 - L, 0), (0, 128 - B, 0)))


def _finalize_kernel(gp_ref, psc_ref, mi_ref, mw_ref):
    tot = None
    for w in range(NW):
        blk = psc_ref[w * 16:(w + 1) * 16, :]
        tot = blk if tot is None else tot + blk
    lane_sum = jnp.sum(tot, axis=1, keepdims=True)
    gsc = lane_sum.reshape(L, B)
    g4 = (gp_ref[0:L, 0:B] + gsc) * (1.0 / T)
    neg = jnp.float32(-jnp.inf)
    g = lax.pad(g4, neg, ((0, 8 - L, 0), (0, 128 - B, 0)))
    rows = lax.broadcasted_iota(jnp.int32, g.shape, 0)
    m1 = jnp.max(g, axis=0, keepdims=True)
    i1 = jnp.min(jnp.where(g == m1, rows, L + 4), axis=0, keepdims=True)
    g2 = jnp.where(rows == i1, neg, g)
    m2 = jnp.max(g2, axis=0, keepdims=True)
    i2 = jnp.min(jnp.where(g2 == m2, rows, L + 4), axis=0, keepdims=True)
    e2 = jnp.exp(m2 - m1)
    w1 = 0.5 / (1.0 + e2)
    w2 = (0.5 * e2) / (1.0 + e2)
    mi_ref[...] = jnp.where(rows == 0, i1, jnp.where(rows == 1, i2, 0))
    mw_ref[...] = jnp.where(rows == 0, w1, jnp.where(rows == 1, w2, 0.0))


_sc_mesh = plsc.VectorSubcoreMesh(core_axis_name="c", subcore_axis_name="s")


@functools.partial(
    pl.kernel,
    out_type=jax.ShapeDtypeStruct((NW * 16, 16), jnp.float32),
    mesh=_sc_mesh,
    scratch_types=[
        pltpu.VMEM((D,), jnp.float32),
        pltpu.VMEM((16, 16), jnp.float32),
        pltpu.VMEM((NR2, B, D), jnp.float32),
        pltpu.VMEM((NR2, B, D), jnp.float32),
        pltpu.SemaphoreType.DMA,
        pltpu.SemaphoreType.DMA,
    ],
)
def _sc_gate(wg_hbm, h0_hbm, h1_hbm, h2_hbm, h3_hbm, psc_hbm,
             wg_v, acc_v, buf0, buf1, sem0, sem1):
    wid = lax.axis_index("s") * NC + lax.axis_index("c")
    base = T_TC + wid * ROWS_W
    pltpu.sync_copy(wg_hbm, wg_v)
    hs = (h0_hbm, h1_hbm, h2_hbm, h3_hbm)
    bufs = (buf0, buf1)
    sems = (sem0, sem1)
    NK = GCHUNKS * L

    for row in range(L * B):
        acc_v[row, pl.ds(0, 16)] = jnp.zeros((16,), jnp.float32)

    def issue(l, c):
        pltpu.async_copy(hs[l].at[pl.ds(base + c * NR2, NR2)],
                         bufs[l % 2], sems[l % 2])

    issue(0, 0)
    issue(1, 0)

    @pl.loop(0, GCHUNKS)
    def _(c):
        for l in range(L):
            k = c * L + l
            buf, sem = bufs[l % 2], sems[l % 2]
            pltpu.make_async_copy(hs[0].at[pl.ds(base, NR2)], buf, sem).wait()
            for b_ in range(B):
                zeros8 = tuple(jnp.zeros((16,), jnp.float32) for _ in range(8))

                @pl.loop(0, NR2, init_carry=zeros8)
                def accs(r, acc, buf=buf, b_=b_):
                    new = []
                    for p in range(8):
                        a = acc[p]
                        for j in range(8):
                            q = (p * 8 + j) * 16
                            a = a + (buf[r, b_, pl.ds(q, 16)]
                                     * wg_v[pl.ds(q, 16)])
                        new.append(a)
                    return tuple(new)

                slab_acc = accs[0]
                for p in range(1, 8):
                    slab_acc = slab_acc + accs[p]
                row = l * B + b_
                acc_v[row, pl.ds(0, 16)] = acc_v[row, pl.ds(0, 16)] + slab_acc

            @pl.when(k + 2 < NK)
            def _(l=l, c=c):
                l2 = (l + 2) % L
                c2 = c + (1 if l >= L - 2 else 0)
                pltpu.async_copy(hs[l2].at[pl.ds(base + c2 * NR2, NR2)],
                                 bufs[l2 % 2], sems[l2 % 2])

    pltpu.sync_copy(acc_v, psc_hbm.at[pl.ds(wid * 16, 16)])


@functools.partial(
    pl.kernel,
    out_type=jax.ShapeDtypeStruct((T, B, D), jnp.float32),
    mesh=_sc_mesh,
    scratch_types=[
        pltpu.VMEM((128,), jnp.int32),
        pltpu.VMEM((128,), jnp.int32),
        pltpu.VMEM((128,), jnp.float32),
        pltpu.VMEM((128,), jnp.float32),
        pltpu.VMEM((NR, D), jnp.float32),
        pltpu.VMEM((NR, D), jnp.float32),
        pltpu.VMEM((NR, D), jnp.float32),
        pltpu.VMEM((NR, D), jnp.float32),
        pltpu.VMEM((NR, D), jnp.float32),
        pltpu.VMEM((NR, D), jnp.float32),
        pltpu.SemaphoreType.DMA,
        pltpu.SemaphoreType.DMA,
        pltpu.SemaphoreType.DMA,
    ],
)
def _sc_combine(mi_hbm, mw_hbm, h0_hbm, h1_hbm, h2_hbm, h3_hbm, out_hbm,
                ia_v, ib_v, wa_v, wb_v, a0, b0, a1, b1, a2, b2,
                sem0, sem1, sem2):
    wid = lax.axis_index("s") * NC + lax.axis_index("c")
    b = wid % B
    tg = wid // B
    base = tg * PER_W
    pltpu.sync_copy(mi_hbm.at[0], ia_v)
    pltpu.sync_copy(mi_hbm.at[1], ib_v)
    pltpu.sync_copy(mw_hbm.at[0], wa_v)
    pltpu.sync_copy(mw_hbm.at[1], wb_v)
    sA = ia_v[pl.ds(b, 16)][0]
    sB = ib_v[pl.ds(b, 16)][0]
    wA = wa_v[pl.ds(b, 16)][0]
    wB = wb_v[pl.ds(b, 16)][0]
    hs = (h0_hbm, h1_hbm, h2_hbm, h3_hbm)

    def issue(t0, bufa, bufb, sem):
        for l in range(L):
            @pl.when(sA == l)
            def _(l=l):
                pltpu.async_copy(hs[l].at[pl.ds(t0, NR), b], bufa, sem)

            @pl.when(sB == l)
            def _(l=l):
                pltpu.async_copy(hs[l].at[pl.ds(t0, NR), b], bufb, sem)

    def drain(bufa, bufb, sem):
        pltpu.make_async_copy(h0_hbm.at[pl.ds(0, NR), b], bufa, sem).wait()
        pltpu.make_async_copy(h0_hbm.at[pl.ds(0, NR), b], bufb, sem).wait()

    def compute(bufa, bufb):
        for r in range(NR):
            @plsc.parallel_loop(0, D // 16, unroll=8)
            def _(j, r=r):
                a = bufa[r, pl.ds(j * 16, 16)]
                bv = bufb[r, pl.ds(j * 16, 16)]
                bufa[r, pl.ds(j * 16, 16)] = wA * a + wB * bv

    slots = ((a0, b0, sem0), (a1, b1, sem1), (a2, b2, sem2))
    issue(base, a0, b0, sem0)
    issue(base + NR, a1, b1, sem1)

    @pl.loop(0, ((CHUNKS + 2) // 3) * 3, step=3)
    def _(c0):
        for s in range(3):
            c = c0 + s
            sa, sb, sem = slots[s]
            na, nb, nsem = slots[(s + 2) % 3]

            @pl.when(c + 2 < CHUNKS)
            def _(na=na, nb=nb, nsem=nsem, c=c):
                issue(base + (c + 2) * NR, na, nb, nsem)

            @pl.when(c < CHUNKS)
            def _(sa=sa, sb=sb, sem=sem, c=c):
                drain(sa, sb, sem)
                compute(sa, sb)
                pltpu.sync_copy(sa, out_hbm.at[pl.ds(base + c * NR, NR), b])


def kernel(h0, h1, h2, h3, Wg, bg):
    del bg
    wg2 = Wg.reshape(1, D)
    h_spec = pl.BlockSpec((TB, B, D), lambda i: (i, 0, 0))
    meta_spec = pl.BlockSpec((8, 128), lambda i: (0, 0))
    psc = _sc_gate(Wg.reshape(D), h0, h1, h2, h3)
    gp = pl.pallas_call(
        _gate_kernel,
        grid=(T_TC // TB,),
        in_specs=[h_spec, h_spec, h_spec, h_spec,
                  pl.BlockSpec((1, D), lambda i: (0, 0))],
        out_specs=meta_spec,
        out_shape=jax.ShapeDtypeStruct((8, 128), jnp.float32),
        scratch_shapes=[pltpu.VMEM((L, B, D), jnp.float32)],
    )(h0, h1, h2, h3, wg2)
    mi, mw = pl.pallas_call(
        _finalize_kernel,
        grid=(1,),
        in_specs=[pl.BlockSpec((8, 128), lambda i: (0, 0)),
                  pl.BlockSpec((NW * 16, 16), lambda i: (0, 0))],
        out_specs=[meta_spec, meta_spec],
        out_shape=[jax.ShapeDtypeStruct((8, 128), jnp.int32),
                   jax.ShapeDtypeStruct((8, 128), jnp.float32)],
    )(gp, psc)
    return _sc_combine(mi, mw, h0, h1, h2, h3)


# speedup vs baseline: 1.7112x; 1.2631x over previous
import functools

import jax
import jax.numpy as jnp
from jax import lax
from jax.experimental import pallas as pl
from jax.experimental.pallas import tpu as pltpu
from jax.experimental.pallas import tpu_sc as plsc

T, B, D, L = 2048, 4, 1024, 4
TB = 128

_SC_INFO = plsc.get_sparse_core_info()
NC, NS = _SC_INFO.num_cores, _SC_INFO.num_subcores
NW = NC * NS
TGROUPS = NW // B
PER_W = T // TGROUPS
NR = 16
CHUNKS = PER_W // NR


def _gate_kernel(h0_ref, h1_ref, h2_ref, h3_ref, wg_ref,
                 mi_ref, mw_ref, acc_ref):
    i = pl.program_id(0)
    nsteps = pl.num_programs(0)

    @pl.when(i == 0)
    def _init():
        acc_ref[...] = jnp.zeros_like(acc_ref)

    for l, h_ref in enumerate((h0_ref, h1_ref, h2_ref, h3_ref)):
        acc_ref[l] += jnp.sum(h_ref[...], axis=0)

    @pl.when(i == nsteps - 1)
    def _finish():
        wgv = wg_ref[...]
        colsum = acc_ref[...]
        gate_lb = jnp.sum(colsum * wgv[None], axis=2) * (1.0 / T)
        neg = jnp.float32(-jnp.inf)
        g = lax.pad(gate_lb, neg, ((0, 8 - L, 0), (0, 128 - B, 0)))
        rows = lax.broadcasted_iota(jnp.int32, g.shape, 0)
        m1 = jnp.max(g, axis=0, keepdims=True)
        i1 = jnp.min(jnp.where(g == m1, rows, L + 4), axis=0, keepdims=True)
        g2 = jnp.where(rows == i1, neg, g)
        m2 = jnp.max(g2, axis=0, keepdims=True)
        i2 = jnp.min(jnp.where(g2 == m2, rows, L + 4), axis=0, keepdims=True)
        e2 = jnp.exp(m2 - m1)
        w1 = 0.5 / (1.0 + e2)
        w2 = (0.5 * e2) / (1.0 + e2)
        mi_ref[...] = jnp.where(rows == 0, i1, jnp.where(rows == 1, i2, 0))
        mw_ref[...] = jnp.where(rows == 0, w1, jnp.where(rows == 1, w2, 0.0))


_sc_mesh = plsc.VectorSubcoreMesh(core_axis_name="c", subcore_axis_name="s")


@functools.partial(
    pl.kernel,
    out_type=jax.ShapeDtypeStruct((T, B, D), jnp.float32),
    mesh=_sc_mesh,
    scratch_types=[
        pltpu.VMEM((128,), jnp.int32),
        pltpu.VMEM((128,), jnp.int32),
        pltpu.VMEM((128,), jnp.float32),
        pltpu.VMEM((128,), jnp.float32),
        pltpu.VMEM((NR, D), jnp.float32),
        pltpu.VMEM((NR, D), jnp.float32),
        pltpu.VMEM((NR, D), jnp.float32),
        pltpu.VMEM((NR, D), jnp.float32),
        pltpu.VMEM((NR, D), jnp.float32),
        pltpu.VMEM((NR, D), jnp.float32),
        pltpu.SemaphoreType.DMA,
        pltpu.SemaphoreType.DMA,
        pltpu.SemaphoreType.DMA,
    ],
)
def _sc_combine(mi_hbm, mw_hbm, h0_hbm, h1_hbm, h2_hbm, h3_hbm, out_hbm,
                ia_v, ib_v, wa_v, wb_v, a0, b0, a1, b1, a2, b2,
                sem0, sem1, sem2):
    wid = lax.axis_index("s") * NC + lax.axis_index("c")
    b = wid % B
    tg = wid // B
    base = tg * PER_W
    pltpu.sync_copy(mi_hbm.at[0], ia_v)
    pltpu.sync_copy(mi_hbm.at[1], ib_v)
    pltpu.sync_copy(mw_hbm.at[0], wa_v)
    pltpu.sync_copy(mw_hbm.at[1], wb_v)
    sA = ia_v[pl.ds(b, 16)][0]
    sB = ib_v[pl.ds(b, 16)][0]
    wA = wa_v[pl.ds(b, 16)][0]
    wB = wb_v[pl.ds(b, 16)][0]
    hs = (h0_hbm, h1_hbm, h2_hbm, h3_hbm)

    def issue(t0, bufa, bufb, sem):
        for l in range(L):
            @pl.when(sA == l)
            def _(l=l):
                pltpu.async_copy(hs[l].at[pl.ds(t0, NR), b], bufa, sem)

            @pl.when(sB == l)
            def _(l=l):
                pltpu.async_copy(hs[l].at[pl.ds(t0, NR), b], bufb, sem)

    def drain(bufa, bufb, sem):
        pltpu.make_async_copy(h0_hbm.at[pl.ds(0, NR), b], bufa, sem).wait()
        pltpu.make_async_copy(h0_hbm.at[pl.ds(0, NR), b], bufb, sem).wait()

    def compute(bufa, bufb):
        for r in range(NR):
            @plsc.parallel_loop(0, D // 16, unroll=8)
            def _(j, r=r):
                a = bufa[r, pl.ds(j * 16, 16)]
                bv = bufb[r, pl.ds(j * 16, 16)]
                bufa[r, pl.ds(j * 16, 16)] = wA * a + wB * bv

    slots = ((a0, b0, sem0), (a1, b1, sem1), (a2, b2, sem2))
    issue(base, a0, b0, sem0)
    issue(base + NR, a1, b1, sem1)

    @pl.loop(0, ((CHUNKS + 2) // 3) * 3, step=3)
    def _(c0):
        for s in range(3):
            c = c0 + s
            sa, sb, sem = slots[s]
            na, nb, nsem = slots[(s + 2) % 3]

            @pl.when(c + 2 < CHUNKS)
            def _(na=na, nb=nb, nsem=nsem, c=c):
                issue(base + (c + 2) * NR, na, nb, nsem)

            @pl.when(c < CHUNKS)
            def _(sa=sa, sb=sb, sem=sem, c=c):
                drain(sa, sb, sem)
                compute(sa, sb)
                pltpu.sync_copy(sa, out_hbm.at[pl.ds(base + c * NR, NR), b])


def kernel(h0, h1, h2, h3, Wg, bg):
    del bg
    wg2 = Wg.reshape(1, D)
    h_spec = pl.BlockSpec((TB, B, D), lambda i: (i, 0, 0))
    meta_spec = pl.BlockSpec((8, 128), lambda i: (0, 0))
    mi, mw = pl.pallas_call(
        _gate_kernel,
        grid=(T // TB,),
        in_specs=[h_spec, h_spec, h_spec, h_spec,
                  pl.BlockSpec((1, D), lambda i: (0, 0))],
        out_specs=[meta_spec, meta_spec],
        out_shape=[jax.ShapeDtypeStruct((8, 128), jnp.int32),
                   jax.ShapeDtypeStruct((8, 128), jnp.float32)],
        scratch_shapes=[pltpu.VMEM((L, B, D), jnp.float32)],
    )(h0, h1, h2, h3, wg2)
    return _sc_combine(mi, mw, h0, h1, h2, h3)
